# baseline (speedup 1.0000x reference)
import jax
import jax.numpy as jnp
from jax.experimental import pallas as pl
from jax.experimental.pallas import tpu as pltpu


def _pe_add_kernel(inp_ref, pos_ref, out_ref, pos_t):
    b = pl.program_id(0)
    h = pl.program_id(1)

    @pl.when(jnp.logical_and(b == 0, h == 0))
    def _():
        pos_t[...] = pos_ref[...].T

    out_ref[...] = inp_ref[...] + pos_t[...]


def kernel(input_tensor, pos_table):
    batch, feature, height, seq = input_tensor.shape

    x2d = input_tensor.reshape(batch * feature, height * seq)
    grid = (batch, height)
    out2d = pl.pallas_call(
        _pe_add_kernel,
        grid=grid,
        in_specs=[
            pl.BlockSpec((feature, seq), lambda b, h: (b, h)),
            pl.BlockSpec((seq, feature), lambda b, h: (0, 0)),
        ],
        out_specs=pl.BlockSpec((feature, seq), lambda b, h: (b, h)),
        out_shape=jax.ShapeDtypeStruct(x2d.shape, x2d.dtype),
        scratch_shapes=[pltpu.VMEM((feature, seq), input_tensor.dtype)],
    )(x2d, pos_table)
    return out2d.reshape(input_tensor.shape)


# speedup vs baseline: 3.6492x; 3.6492x over previous
import jax
import jax.numpy as jnp
from jax.experimental import pallas as pl
from jax.experimental.pallas import tpu as pltpu

_FBLK = 8


def _pe_add_kernel(inp_ref, pos_ref, out_ref, pos_t):
    b = pl.program_id(0)
    g = pl.program_id(1)

    @pl.when(jnp.logical_and(b == 0, g == 0))
    def _():
        pos_t[...] = pos_ref[...].T

    slab = pos_t[pl.ds(g * _FBLK, _FBLK), :]
    out_ref[...] = inp_ref[...] + slab[None, :, None, :]


def kernel(input_tensor, pos_table):
    batch, feature, height, seq = input_tensor.shape

    grid = (batch, feature // _FBLK)
    return pl.pallas_call(
        _pe_add_kernel,
        grid=grid,
        in_specs=[
            pl.BlockSpec((1, _FBLK, height, seq), lambda b, g: (b, g, 0, 0)),
            pl.BlockSpec((seq, feature), lambda b, g: (0, 0)),
        ],
        out_specs=pl.BlockSpec((1, _FBLK, height, seq), lambda b, g: (b, g, 0, 0)),
        out_shape=jax.ShapeDtypeStruct(input_tensor.shape, input_tensor.dtype),
        scratch_shapes=[pltpu.VMEM((feature, seq), input_tensor.dtype)],
    )(input_tensor, pos_table)


# speedup vs baseline: 3.9891x; 1.0931x over previous
import jax
import jax.numpy as jnp
from jax.experimental import pallas as pl
from jax.experimental.pallas import tpu as pltpu

_FBLK = 16


def _pe_add_kernel(inp_ref, pos_ref, out_ref, pos_t):
    b = pl.program_id(0)
    g = pl.program_id(1)

    @pl.when(jnp.logical_and(b == 0, g == 0))
    def _():
        pos_t[...] = pos_ref[...].T

    slab = pos_t[pl.ds(g * _FBLK, _FBLK), :]
    out_ref[...] = inp_ref[...] + slab[None, :, None, :]


def kernel(input_tensor, pos_table):
    batch, feature, height, seq = input_tensor.shape

    grid = (batch, feature // _FBLK)
    return pl.pallas_call(
        _pe_add_kernel,
        grid=grid,
        in_specs=[
            pl.BlockSpec((1, _FBLK, height, seq), lambda b, g: (b, g, 0, 0)),
            pl.BlockSpec((seq, feature), lambda b, g: (0, 0)),
        ],
        out_specs=pl.BlockSpec((1, _FBLK, height, seq), lambda b, g: (b, g, 0, 0)),
        out_shape=jax.ShapeDtypeStruct(input_tensor.shape, input_tensor.dtype),
        scratch_shapes=[pltpu.VMEM((feature, seq), input_tensor.dtype)],
    )(input_tensor, pos_table)


# speedup vs baseline: 4.0802x; 1.0228x over previous
import jax
import jax.numpy as jnp
from jax.experimental import pallas as pl
from jax.experimental.pallas import tpu as pltpu

_FBLK = 32


def _pe_add_kernel(inp_ref, pos_ref, out_ref, pos_t):
    b = pl.program_id(0)
    g = pl.program_id(1)

    @pl.when(jnp.logical_and(b == 0, g == 0))
    def _():
        pos_t[...] = pos_ref[...].T

    slab = pos_t[pl.ds(g * _FBLK, _FBLK), :]
    out_ref[...] = inp_ref[...] + slab[None, :, None, :]


def kernel(input_tensor, pos_table):
    batch, feature, height, seq = input_tensor.shape

    grid = (batch, feature // _FBLK)
    return pl.pallas_call(
        _pe_add_kernel,
        grid=grid,
        in_specs=[
            pl.BlockSpec((1, _FBLK, height, seq), lambda b, g: (b, g, 0, 0)),
            pl.BlockSpec((seq, feature), lambda b, g: (0, 0)),
        ],
        out_specs=pl.BlockSpec((1, _FBLK, height, seq), lambda b, g: (b, g, 0, 0)),
        out_shape=jax.ShapeDtypeStruct(input_tensor.shape, input_tensor.dtype),
        scratch_shapes=[pltpu.VMEM((feature, seq), input_tensor.dtype)],
    )(input_tensor, pos_table)
